# baseline (speedup 1.0000x reference)
import jax
import jax.numpy as jnp
from jax import lax
from jax.experimental import pallas as pl
from jax.experimental.pallas import tpu as pltpu
from jax.experimental.pallas import tpu_sc as plsc

BATCH = 16384
CHANNELS = 1024
DUP = 64
LANES = 16
NUM_CORES = 2
NUM_SUBCORES = 16
NUM_WORKERS = NUM_CORES * NUM_SUBCORES
ROWS_PER_WORKER = BATCH // NUM_WORKERS
CHUNK = 32
NUM_CHUNKS = ROWS_PER_WORKER // CHUNK
JBLOCKS = CHANNELS // LANES
DUP_JBLOCKS = DUP // LANES


def _mask_plain(v, mn, mx):
    t = jnp.where(v >= mn, v, 0.0)
    return jnp.where(t <= mx, t, 0.0)


def _corrected(v1, v2, mn, mx):
    v1 = jnp.where(v1 != v1, 0.0, v1)
    v2 = jnp.where(v2 != v2, 0.0, v2)
    in1 = (v1 >= mn) & (v1 <= mx)
    in2 = (v2 >= mn) & (v2 <= mx)
    return jnp.where(
        in1 & in2,
        jnp.minimum(v1, v2),
        jnp.where(in2, v2, jnp.where(in1, v1, 0.0)),
    )


def _body(x_hbm, mn_hbm, mx_hbm, out_hbm, buf, mn_v, mx_v):
    wid = lax.axis_index("s") * NUM_CORES + lax.axis_index("c")
    base = wid * ROWS_PER_WORKER
    pltpu.sync_copy(mn_hbm, mn_v)
    pltpu.sync_copy(mx_hbm, mx_v)

    def chunk_body(i, carry):
        r0 = base + i * CHUNK
        pltpu.sync_copy(x_hbm.at[pl.ds(r0, CHUNK), :], buf)

        for j in range(DUP_JBLOCKS):
            mn = mn_v[pl.ds(j * LANES, LANES)]
            mx = mx_v[pl.ds(j * LANES, LANES)]

            def row_corr(r, c, j=j, mn=mn, mx=mx):
                v1 = buf[r, pl.ds(j * LANES, LANES)]
                v2 = buf[r, pl.ds(DUP + j * LANES, LANES)]
                buf[r, pl.ds(j * LANES, LANES)] = _corrected(v1, v2, mn, mx)
                return c

            lax.fori_loop(0, CHUNK, row_corr, 0)

        for j in range(DUP_JBLOCKS, JBLOCKS):
            mn = mn_v[pl.ds(j * LANES, LANES)]
            mx = mx_v[pl.ds(j * LANES, LANES)]

            def row_plain(r, c, j=j, mn=mn, mx=mx):
                v = buf[r, pl.ds(j * LANES, LANES)]
                buf[r, pl.ds(j * LANES, LANES)] = _mask_plain(v, mn, mx)
                return c

            lax.fori_loop(0, CHUNK, row_plain, 0)

        pltpu.sync_copy(buf, out_hbm.at[pl.ds(r0, CHUNK), :])
        return carry

    lax.fori_loop(0, NUM_CHUNKS, chunk_body, 0)


def kernel(x, min_val, max_val):
    mesh = plsc.VectorSubcoreMesh(
        core_axis_name="c", subcore_axis_name="s",
        num_cores=NUM_CORES, num_subcores=NUM_SUBCORES,
    )
    f = pl.kernel(
        _body,
        out_type=jax.ShapeDtypeStruct((BATCH, CHANNELS), jnp.float32),
        mesh=mesh,
        scratch_types=[
            pltpu.VMEM((CHUNK, CHANNELS), jnp.float32),
            pltpu.VMEM((CHANNELS,), jnp.float32),
            pltpu.VMEM((CHANNELS,), jnp.float32),
        ],
    )
    return f(x, min_val, max_val)


# speedup vs baseline: 2.2015x; 2.2015x over previous
import jax
import jax.numpy as jnp
from jax import lax
from jax.experimental import pallas as pl
from jax.experimental.pallas import tpu as pltpu
from jax.experimental.pallas import tpu_sc as plsc

BATCH = 16384
CHANNELS = 1024
DUP = 64
LANES = 16
NUM_CORES = 2
NUM_SUBCORES = 16
NUM_WORKERS = NUM_CORES * NUM_SUBCORES
ROWS_PER_WORKER = BATCH // NUM_WORKERS
CHUNK = 16
NUM_CHUNKS = ROWS_PER_WORKER // CHUNK
PAIRS = NUM_CHUNKS // 2
JBLOCKS = CHANNELS // LANES
DUP_JBLOCKS = DUP // LANES
UNROLL = 8


def _mask_plain(v, mn, mx):
    t = jnp.where(v >= mn, v, 0.0)
    return jnp.where(t <= mx, t, 0.0)


def _corrected(v1, v2, mn, mx):
    v1 = jnp.where(v1 != v1, 0.0, v1)
    v2 = jnp.where(v2 != v2, 0.0, v2)
    in1 = (v1 >= mn) & (v1 <= mx)
    in2 = (v2 >= mn) & (v2 <= mx)
    return jnp.where(
        in1 & in2,
        jnp.minimum(v1, v2),
        jnp.where(in2, v2, jnp.where(in1, v1, 0.0)),
    )


def _compute(src, dst, mn_v, mx_v):
    for j in range(DUP_JBLOCKS):
        mn = mn_v[pl.ds(j * LANES, LANES)]
        mx = mx_v[pl.ds(j * LANES, LANES)]

        @plsc.parallel_loop(0, CHUNK, unroll=UNROLL)
        def _(r, j=j, mn=mn, mx=mx):
            v1 = src[r, pl.ds(j * LANES, LANES)]
            v2 = src[r, pl.ds(DUP + j * LANES, LANES)]
            dst[r, pl.ds(j * LANES, LANES)] = _corrected(v1, v2, mn, mx)

    for j in range(DUP_JBLOCKS, JBLOCKS):
        mn = mn_v[pl.ds(j * LANES, LANES)]
        mx = mx_v[pl.ds(j * LANES, LANES)]

        @plsc.parallel_loop(0, CHUNK, unroll=UNROLL)
        def _(r, j=j, mn=mn, mx=mx):
            v = src[r, pl.ds(j * LANES, LANES)]
            dst[r, pl.ds(j * LANES, LANES)] = _mask_plain(v, mn, mx)


def _body(x_hbm, mn_hbm, mx_hbm, out_hbm,
          in0, in1, out0, out1, mn_v, mx_v,
          semL0, semL1, semS0, semS1):
    wid = lax.axis_index("s") * NUM_CORES + lax.axis_index("c")
    base = wid * ROWS_PER_WORKER
    pltpu.sync_copy(mn_hbm, mn_v)
    pltpu.sync_copy(mx_hbm, mx_v)

    pltpu.async_copy(x_hbm.at[pl.ds(base, CHUNK), :], in0, semL0)
    pltpu.async_copy(x_hbm.at[pl.ds(base + CHUNK, CHUNK), :], in1, semL1)

    def pair_body(i, carry):
        for ib, ob, semL, semS, off in (
            (in0, out0, semL0, semS0, 0),
            (in1, out1, semL1, semS1, 1),
        ):
            r0 = base + (2 * i + off) * CHUNK
            pltpu.make_async_copy(
                x_hbm.at[pl.ds(r0, CHUNK), :], ib, semL).wait()

            @pl.when(i > 0)
            def _():
                pltpu.make_async_copy(
                    ob, out_hbm.at[pl.ds(r0, CHUNK), :], semS).wait()

            _compute(ib, ob, mn_v, mx_v)
            pltpu.async_copy(ob, out_hbm.at[pl.ds(r0, CHUNK), :], semS)

            @pl.when(i < PAIRS - 1)
            def _():
                pltpu.async_copy(
                    x_hbm.at[pl.ds(r0 + 2 * CHUNK, CHUNK), :], ib, semL)
        return carry

    lax.fori_loop(0, PAIRS, pair_body, 0)

    pltpu.make_async_copy(out0, out_hbm.at[pl.ds(base, CHUNK), :], semS0).wait()
    pltpu.make_async_copy(out1, out_hbm.at[pl.ds(base, CHUNK), :], semS1).wait()


def kernel(x, min_val, max_val):
    mesh = plsc.VectorSubcoreMesh(
        core_axis_name="c", subcore_axis_name="s",
        num_cores=NUM_CORES, num_subcores=NUM_SUBCORES,
    )
    f = pl.kernel(
        _body,
        out_type=jax.ShapeDtypeStruct((BATCH, CHANNELS), jnp.float32),
        mesh=mesh,
        scratch_types=[
            pltpu.VMEM((CHUNK, CHANNELS), jnp.float32),
            pltpu.VMEM((CHUNK, CHANNELS), jnp.float32),
            pltpu.VMEM((CHUNK, CHANNELS), jnp.float32),
            pltpu.VMEM((CHUNK, CHANNELS), jnp.float32),
            pltpu.VMEM((CHANNELS,), jnp.float32),
            pltpu.VMEM((CHANNELS,), jnp.float32),
            pltpu.SemaphoreType.DMA,
            pltpu.SemaphoreType.DMA,
            pltpu.SemaphoreType.DMA,
            pltpu.SemaphoreType.DMA,
        ],
    )
    return f(x, min_val, max_val)
